# baseline (speedup 1.0000x reference)
import dataclasses
import functools

import jax
import jax.numpy as jnp
from jax import lax
from jax.experimental import pallas as pl
from jax.experimental.pallas import tpu as pltpu
from jax.experimental.pallas import tpu_sc as plsc

N = 10000
IN_CH = 128
HID = 128
N_CLASSES = 7
E = 320000

NC = 2
NS = 16
NW = NC * NS
CHUNK = 128
CH = 80
NCHUNKS = NW * CH
IDX_ROWS = NCHUNKS
E_PAD = NCHUNKS * CHUNK
ROWS_PER_SUB = 640
N_ACC = NS * ROWS_PER_SUB
C_PAD = 16

_mesh = plsc.VectorSubcoreMesh(core_axis_name="c", subcore_axis_name="s")

_sc_params = pltpu.CompilerParams()
if "needs_layout_passes" in pltpu.CompilerParams.__dataclass_fields__:
    _sc_params = dataclasses.replace(_sc_params, needs_layout_passes=False)


def _sc_agg(h, src3, dst3):
    @functools.partial(
        pl.kernel,
        mesh=_mesh,
        out_type=jax.ShapeDtypeStruct((NC, N_ACC, HID), jnp.float32),
        scratch_types=[
            pltpu.VMEM((CH, CHUNK), jnp.int32),
            pltpu.VMEM((CH, CHUNK), jnp.int32),
            pltpu.VMEM((CHUNK, HID), jnp.float32),
            pltpu.VMEM_SHARED((N_ACC, HID), jnp.float32),
        ],
    )
    def body(h_hbm, src_hbm, dst_hbm, out_hbm, src_v, dst_v, rows_v, acc_sh):
        c = lax.axis_index("c")
        s = lax.axis_index("s")
        @pl.loop(0, CHUNK)
        def _(i):
            @pl.loop(0, HID, step=16)
            def _(k):
                rows_v[i, pl.ds(k, 16)] = jnp.zeros((16,), jnp.float32)

        @pl.loop(0, ROWS_PER_SUB // CHUNK)
        def _(i):
            pltpu.sync_copy(
                rows_v, acc_sh.at[pl.ds(s * ROWS_PER_SUB + i * CHUNK, CHUNK)]
            )

        base = (s * NC + c) * CH
        pltpu.sync_copy(src_hbm.at[pl.ds(base, CH)], src_v)
        pltpu.sync_copy(dst_hbm.at[pl.ds(base, CH)], dst_v)
        plsc.subcore_barrier()

        @pl.loop(0, CH)
        def _(j):
            pltpu.sync_copy(h_hbm.at[src_v.at[j]], rows_v)
            pltpu.sync_copy(rows_v, acc_sh.at[dst_v.at[j]], add=True)

        plsc.subcore_barrier()
        pltpu.sync_copy(
            acc_sh.at[pl.ds(s * ROWS_PER_SUB, ROWS_PER_SUB)],
            out_hbm.at[c].at[pl.ds(s * ROWS_PER_SUB, ROWS_PER_SUB)],
        )

    return body(h, src3, dst3)


def _sc_deg(dst3):
    @functools.partial(
        pl.kernel,
        mesh=_mesh,
        out_type=jax.ShapeDtypeStruct((NW, N_ACC), jnp.float32),
        compiler_params=_sc_params,
        scratch_types=[
            pltpu.VMEM((CH, CHUNK), jnp.int32),
            pltpu.VMEM((N_ACC,), jnp.float32),
        ],
    )
    def body(dst_hbm, out_hbm, dst_v, hist_v):
        c = lax.axis_index("c")
        s = lax.axis_index("s")
        wid = s * NC + c

        @pl.loop(0, N_ACC, step=16)
        def _(k):
            hist_v[pl.ds(k, 16)] = jnp.zeros((16,), jnp.float32)

        ones = jnp.ones((16,), jnp.float32)
        base = wid * CH
        pltpu.sync_copy(dst_hbm.at[pl.ds(base, CH)], dst_v)

        @pl.loop(0, CH)
        def _(j):
            @pl.loop(0, CHUNK, step=16)
            def _(k):
                idx = dst_v[j, pl.ds(k, 16)]
                plsc.addupdate_scatter(hist_v, [idx], ones)

        pltpu.sync_copy(hist_v, out_hbm.at[wid])

    return body(dst3)


_R_BLK = 2000
_GRID = N // _R_BLK


def _mm1_body(x_ref, w_ref, o_ref):
    o_ref[...] = jnp.dot(
        x_ref[...], w_ref[...], preferred_element_type=jnp.float32
    )


def _scale1_body(h_ref, deg_ref, hp_ref, dinv_ref):
    deg = jnp.sum(deg_ref[...], axis=1)[:, None] + 1.0
    dinv = lax.rsqrt(deg)
    dinv_ref[...] = dinv
    hp_ref[...] = h_ref[...] * dinv


def _combine1_body(p0_ref, p1_ref, hp_ref, dinv_ref, b1_ref, o_ref):
    dinv = dinv_ref[...]
    t = (p0_ref[...] + p1_ref[...] + hp_ref[...]) * dinv + b1_ref[...]
    o_ref[...] = jnp.maximum(t, 0.0) * dinv


def _final_body(q0_ref, q1_ref, u_ref, dinv_ref, w2_ref, b2_ref, o_ref):
    t = (q0_ref[...] + q1_ref[...] + u_ref[...]) * dinv_ref[...]
    logits = (
        jnp.dot(t, w2_ref[...], preferred_element_type=jnp.float32)
        + b2_ref[...]
    )
    col = lax.broadcasted_iota(jnp.int32, logits.shape, 1)
    logits = jnp.where(col < N_CLASSES, logits, -jnp.inf)
    m = jnp.max(logits, axis=1, keepdims=True)
    lse = jnp.log(jnp.sum(jnp.exp(logits - m), axis=1, keepdims=True)) + m
    o_ref[...] = logits - lse


def _row_spec(width):
    return pl.BlockSpec((_R_BLK, width), lambda i: (i, 0))


def _full_spec(shape):
    return pl.BlockSpec(shape, lambda i: (0, 0))


def kernel(x, edge_index, W1, b1, W2, b2):
    ei = edge_index.astype(jnp.int32)
    src = ei[0]
    dst = ei[1]
    pad = IDX_ROWS * CHUNK - E
    srcp = jnp.concatenate([src, jnp.zeros((pad,), jnp.int32)])
    pad_dst = N + (jnp.arange(pad, dtype=jnp.int32) % 128)
    dstp = jnp.concatenate([dst, pad_dst])
    src3 = srcp.reshape(IDX_ROWS, CHUNK)
    dst3 = dstp.reshape(IDX_ROWS, CHUNK)

    W2p = jnp.zeros((HID, C_PAD), jnp.float32).at[:, :N_CLASSES].set(W2)
    b1r = b1.reshape(1, HID)
    b2r = jnp.zeros((1, C_PAD), jnp.float32).at[0, :N_CLASSES].set(b2)

    degs = _sc_deg(dst3)[:, :N].T

    h1 = pl.pallas_call(
        _mm1_body,
        grid=(_GRID,),
        in_specs=[_row_spec(IN_CH), _full_spec((IN_CH, HID))],
        out_specs=_row_spec(HID),
        out_shape=jax.ShapeDtypeStruct((N, HID), jnp.float32),
    )(x, W1)

    h1p, dinv = pl.pallas_call(
        _scale1_body,
        grid=(_GRID,),
        in_specs=[_row_spec(HID), pl.BlockSpec((_R_BLK, NW), lambda i: (i, 0))],
        out_specs=[_row_spec(HID), _row_spec(1)],
        out_shape=[
            jax.ShapeDtypeStruct((N, HID), jnp.float32),
            jax.ShapeDtypeStruct((N, 1), jnp.float32),
        ],
    )(h1, degs)

    part1 = _sc_agg(h1p, src3, dst3)

    u = pl.pallas_call(
        _combine1_body,
        grid=(_GRID,),
        in_specs=[
            _row_spec(HID),
            _row_spec(HID),
            _row_spec(HID),
            _row_spec(1),
            _full_spec((1, HID)),
        ],
        out_specs=_row_spec(HID),
        out_shape=jax.ShapeDtypeStruct((N, HID), jnp.float32),
    )(part1[0, :N], part1[1, :N], h1p, dinv, b1r)

    part2 = _sc_agg(u, src3, dst3)

    outp = pl.pallas_call(
        _final_body,
        grid=(_GRID,),
        in_specs=[
            _row_spec(HID),
            _row_spec(HID),
            _row_spec(HID),
            _row_spec(1),
            _full_spec((HID, C_PAD)),
            _full_spec((1, C_PAD)),
        ],
        out_specs=_row_spec(C_PAD),
        out_shape=jax.ShapeDtypeStruct((N, C_PAD), jnp.float32),
    )(part2[0, :N], part2[1, :N], u, dinv, W2p, b2r)

    return outp[:, :N_CLASSES]


# speedup vs baseline: 1.4820x; 1.4820x over previous
import dataclasses
import functools

import jax
import jax.numpy as jnp
from jax import lax
from jax.experimental import pallas as pl
from jax.experimental.pallas import tpu as pltpu
from jax.experimental.pallas import tpu_sc as plsc

N = 10000
IN_CH = 128
HID = 128
N_CLASSES = 7
E = 320000

NC = 2
NS = 16
NW = NC * NS
CHUNK = 128
CH = 79
E_PAD = NW * CH * CHUNK
ROWS_PER_SUB = 640
N_ACC = NS * ROWS_PER_SUB
C_PAD = 16

_mesh = plsc.VectorSubcoreMesh(core_axis_name="c", subcore_axis_name="s")

_sc_params = pltpu.CompilerParams()
if "needs_layout_passes" in pltpu.CompilerParams.__dataclass_fields__:
    _sc_params = dataclasses.replace(_sc_params, needs_layout_passes=False)


def _sc_agg(h, src3, dst3):
    @functools.partial(
        pl.kernel,
        mesh=_mesh,
        out_type=jax.ShapeDtypeStruct((NC, N_ACC, HID), jnp.float32),
        scratch_types=[
            pltpu.VMEM((CH, CHUNK), jnp.int32),
            pltpu.VMEM((CH, CHUNK), jnp.int32),
            pltpu.VMEM((CHUNK, HID), jnp.float32),
            pltpu.VMEM_SHARED((N_ACC, HID), jnp.float32),
        ],
    )
    def body(h_hbm, src_hbm, dst_hbm, out_hbm, src_v, dst_v, rows_v, acc_sh):
        c = lax.axis_index("c")
        s = lax.axis_index("s")
        @pl.loop(0, CHUNK)
        def _(i):
            @pl.loop(0, HID, step=16)
            def _(k):
                rows_v[i, pl.ds(k, 16)] = jnp.zeros((16,), jnp.float32)

        @pl.loop(0, ROWS_PER_SUB // CHUNK)
        def _(i):
            pltpu.sync_copy(
                rows_v, acc_sh.at[pl.ds(s * ROWS_PER_SUB + i * CHUNK, CHUNK)]
            )

        wid = s * NC + c
        pltpu.sync_copy(src_hbm.at[wid], src_v)
        pltpu.sync_copy(dst_hbm.at[wid], dst_v)
        plsc.subcore_barrier()

        @pl.loop(0, CH)
        def _(j):
            pltpu.sync_copy(h_hbm.at[src_v.at[j]], rows_v)
            pltpu.sync_copy(rows_v, acc_sh.at[dst_v.at[j]], add=True)

        plsc.subcore_barrier()
        pltpu.sync_copy(
            acc_sh.at[pl.ds(s * ROWS_PER_SUB, ROWS_PER_SUB)],
            out_hbm.at[c].at[pl.ds(s * ROWS_PER_SUB, ROWS_PER_SUB)],
        )

    return body(h, src3, dst3)


def _sc_deg(dst3):
    @functools.partial(
        pl.kernel,
        mesh=_mesh,
        out_type=jax.ShapeDtypeStruct((NW, N_ACC), jnp.float32),
        compiler_params=_sc_params,
        scratch_types=[
            pltpu.VMEM((CH, CHUNK), jnp.int32),
            pltpu.VMEM((N_ACC,), jnp.float32),
        ],
    )
    def body(dst_hbm, out_hbm, dst_v, hist_v):
        c = lax.axis_index("c")
        s = lax.axis_index("s")
        wid = s * NC + c

        @pl.loop(0, N_ACC, step=16)
        def _(k):
            hist_v[pl.ds(k, 16)] = jnp.zeros((16,), jnp.float32)

        ones = jnp.ones((16,), jnp.float32)
        pltpu.sync_copy(dst_hbm.at[wid], dst_v)

        @pl.loop(0, CH)
        def _(j):
            @pl.loop(0, CHUNK, step=16)
            def _(k):
                idx = dst_v[j, pl.ds(k, 16)]
                plsc.addupdate_scatter(hist_v, [idx], ones)

        pltpu.sync_copy(hist_v, out_hbm.at[wid])

    return body(dst3)


_R_BLK = 2000
_GRID = N // _R_BLK


def _mm1_body(x_ref, w_ref, o_ref):
    o_ref[...] = jnp.dot(
        x_ref[...], w_ref[...], preferred_element_type=jnp.float32
    )


def _scale1_body(h_ref, deg_ref, hp_ref, dinv_ref):
    deg = jnp.sum(deg_ref[...], axis=1)[:, None] + 1.0
    dinv = lax.rsqrt(deg)
    dinv_ref[...] = dinv
    hp_ref[...] = h_ref[...] * dinv


def _combine1_body(p0_ref, p1_ref, hp_ref, dinv_ref, b1_ref, o_ref):
    dinv = dinv_ref[...]
    t = (p0_ref[...] + p1_ref[...] + hp_ref[...]) * dinv + b1_ref[...]
    o_ref[...] = jnp.maximum(t, 0.0) * dinv


def _final_body(q0_ref, q1_ref, u_ref, dinv_ref, w2_ref, b2_ref, o_ref):
    t = (q0_ref[...] + q1_ref[...] + u_ref[...]) * dinv_ref[...]
    logits = (
        jnp.dot(t, w2_ref[...], preferred_element_type=jnp.float32)
        + b2_ref[...]
    )
    col = lax.broadcasted_iota(jnp.int32, logits.shape, 1)
    logits = jnp.where(col < N_CLASSES, logits, -jnp.inf)
    m = jnp.max(logits, axis=1, keepdims=True)
    lse = jnp.log(jnp.sum(jnp.exp(logits - m), axis=1, keepdims=True)) + m
    o_ref[...] = logits - lse


def _row_spec(width):
    return pl.BlockSpec((_R_BLK, width), lambda i: (i, 0))


def _full_spec(shape):
    return pl.BlockSpec(shape, lambda i: (0, 0))


def kernel(x, edge_index, W1, b1, W2, b2):
    ei = edge_index.astype(jnp.int32)
    src = ei[0]
    dst = ei[1]
    pad = E_PAD - E
    srcp = jnp.concatenate([src, jnp.zeros((pad,), jnp.int32)])
    pad_dst = N + (jnp.arange(pad, dtype=jnp.int32) % 128)
    dstp = jnp.concatenate([dst, pad_dst])
    src3 = srcp.reshape(NW, CH, CHUNK)
    dst3 = dstp.reshape(NW, CH, CHUNK)

    W2p = jnp.zeros((HID, C_PAD), jnp.float32).at[:, :N_CLASSES].set(W2)
    b1r = b1.reshape(1, HID)
    b2r = jnp.zeros((1, C_PAD), jnp.float32).at[0, :N_CLASSES].set(b2)

    degs = _sc_deg(dst3)[:, :N].T

    h1 = pl.pallas_call(
        _mm1_body,
        grid=(_GRID,),
        in_specs=[_row_spec(IN_CH), _full_spec((IN_CH, HID))],
        out_specs=_row_spec(HID),
        out_shape=jax.ShapeDtypeStruct((N, HID), jnp.float32),
    )(x, W1)

    h1p, dinv = pl.pallas_call(
        _scale1_body,
        grid=(_GRID,),
        in_specs=[_row_spec(HID), pl.BlockSpec((_R_BLK, NW), lambda i: (i, 0))],
        out_specs=[_row_spec(HID), _row_spec(1)],
        out_shape=[
            jax.ShapeDtypeStruct((N, HID), jnp.float32),
            jax.ShapeDtypeStruct((N, 1), jnp.float32),
        ],
    )(h1, degs)

    part1 = _sc_agg(h1p, src3, dst3)

    u = pl.pallas_call(
        _combine1_body,
        grid=(_GRID,),
        in_specs=[
            _row_spec(HID),
            _row_spec(HID),
            _row_spec(HID),
            _row_spec(1),
            _full_spec((1, HID)),
        ],
        out_specs=_row_spec(HID),
        out_shape=jax.ShapeDtypeStruct((N, HID), jnp.float32),
    )(part1[0, :N], part1[1, :N], h1p, dinv, b1r)

    part2 = _sc_agg(u, src3, dst3)

    outp = pl.pallas_call(
        _final_body,
        grid=(_GRID,),
        in_specs=[
            _row_spec(HID),
            _row_spec(HID),
            _row_spec(HID),
            _row_spec(1),
            _full_spec((HID, C_PAD)),
            _full_spec((1, C_PAD)),
        ],
        out_specs=_row_spec(C_PAD),
        out_shape=jax.ShapeDtypeStruct((N, C_PAD), jnp.float32),
    )(part2[0, :N], part2[1, :N], u, dinv, W2p, b2r)

    return outp[:, :N_CLASSES]
